# baseline (speedup 1.0000x reference)
import functools

import jax
import jax.numpy as jnp
from jax.experimental import pallas as pl
from jax.experimental.pallas import tpu as pltpu

_NUM_CORRECTION_LAYERS = 10
_CORRECTION_ALPHA = 0.979
_NUM_SMOOTHING_LAYERS = 10
_SMOOTHING_ALPHA = 0.756

_BE = 2048
_W = 256


def _seg_kernel(dst_ref, gath_ref, out_ref):
    i = pl.program_id(0)

    @pl.when(i == 0)
    def _init():
        out_ref[...] = jnp.zeros_like(out_ref)

    base = dst_ref[0, 0]

    @pl.when(base >= 0)
    def _work():
        rel = dst_ref[0, :] - base
        maxr = jnp.max(rel)
        gath = gath_ref[...]

        nwin = maxr // _W + jnp.int32(1)

        def body(t, carry):
            t = t.astype(jnp.int32) if hasattr(t, "astype") else t
            start = base + t * _W
            iota = jax.lax.broadcasted_iota(jnp.int32, (_W, _BE), 0)
            sel = (rel[None, :] - t * _W) == iota
            contrib = jnp.dot(sel.astype(jnp.float32), gath,
                              preferred_element_type=jnp.float32)
            out_ref[pl.ds(start, _W), :] += contrib
            return carry

        jax.lax.fori_loop(jnp.int32(0), nwin, body, jnp.float32(0.0))


def _upd_kernel(alpha, lo, hi, agg_ref, last_ref, norm_ref, y_ref, h_ref):
    norm = norm_ref[...]
    y = last_ref[...] + alpha * (agg_ref[...] * norm)
    y = jnp.clip(y, lo, hi)
    y_ref[...] = y
    h_ref[...] = y * norm


def _segment_sum(dst2d, gath, n_pad, c):
    e_pad = dst2d.shape[1]
    return pl.pallas_call(
        _seg_kernel,
        grid=(e_pad // _BE,),
        in_specs=[
            pl.BlockSpec((1, _BE), lambda i: (jnp.int32(0), i)),
            pl.BlockSpec((_BE, c), lambda i: (i, jnp.int32(0))),
        ],
        out_specs=pl.BlockSpec((n_pad, c), lambda i: (jnp.int32(0), jnp.int32(0))),
        out_shape=jax.ShapeDtypeStruct((n_pad, c), jnp.float32),
        compiler_params=pltpu.CompilerParams(
            dimension_semantics=("arbitrary",)),
    )(dst2d, gath)


def _update(alpha, lo, hi, agg, last, norm, bn):
    n, c = agg.shape
    kfn = functools.partial(_upd_kernel, alpha, lo, hi)
    return pl.pallas_call(
        kfn,
        grid=(n // bn,),
        in_specs=[
            pl.BlockSpec((bn, c), lambda i: (i, jnp.int32(0))),
            pl.BlockSpec((bn, c), lambda i: (i, jnp.int32(0))),
            pl.BlockSpec((bn, 1), lambda i: (i, jnp.int32(0))),
        ],
        out_specs=[
            pl.BlockSpec((bn, c), lambda i: (i, jnp.int32(0))),
            pl.BlockSpec((bn, c), lambda i: (i, jnp.int32(0))),
        ],
        out_shape=[
            jax.ShapeDtypeStruct((n, c), jnp.float32),
            jax.ShapeDtypeStruct((n, c), jnp.float32),
        ],
    )(agg, last, norm)


def _label_prop(src_pad, dst2d, labels, norm, num_layers, alpha, lo, hi, bn):
    n, c = labels.shape
    n_pad = n + _W
    last = (1.0 - alpha) * labels
    y = labels
    h = y * norm
    for _ in range(num_layers):
        gath = h[src_pad]
        agg = _segment_sum(dst2d, gath, n_pad, c)[:n]
        y, h = _update(alpha, lo, hi, agg, last, norm, bn)
    return y


@jax.jit
def kernel(y_soft, edge_index, y_true, mask):
    n, c = y_soft.shape
    numel = mask.shape[0]
    y_soft = y_soft.astype(jnp.float32)

    src = edge_index[0]
    dst = edge_index[1]
    order = jnp.argsort(dst)
    src_s = src[order].astype(jnp.int32)
    dst_s = dst[order].astype(jnp.int32)

    e = src_s.shape[0]
    e_pad = ((e + _BE - 1) // _BE) * _BE
    pad = e_pad - e
    src_pad = jnp.concatenate([src_s, jnp.zeros((pad,), jnp.int32)])
    dst_pad = jnp.concatenate([dst_s, jnp.full((pad,), -1, jnp.int32)])
    dst2d = dst_pad.reshape(1, e_pad)

    degs = jnp.clip(jnp.bincount(dst, length=n).astype(jnp.float32), 1.0, None)
    norm = (degs ** -0.5)[:, None]

    bn = 4000 if n % 4000 == 0 else n

    y_true_oh = jax.nn.one_hot(y_true, c, dtype=jnp.float32)

    error = jnp.zeros_like(y_soft)
    error = error.at[mask].set(y_true_oh - y_soft[mask])
    smoothed_error = _label_prop(
        src_pad, dst2d, error, norm,
        _NUM_CORRECTION_LAYERS, _CORRECTION_ALPHA, -1.0, 1.0, bn)
    sigma = jnp.abs(error[mask]).sum() / numel
    denom = jnp.abs(smoothed_error).sum(axis=1, keepdims=True)
    scale = sigma / denom
    scale = jnp.where(jnp.isinf(scale) | (scale > 1000.0), 1.0, scale)
    y = y_soft + scale * smoothed_error

    y = y.at[mask].set(y_true_oh)
    y = _label_prop(
        src_pad, dst2d, y, norm,
        _NUM_SMOOTHING_LAYERS, _SMOOTHING_ALPHA, 0.0, 1.0, bn)
    return y
